# Initial kernel scaffold; baseline (speedup 1.0000x reference)
import jax
import jax.numpy as jnp
from jax.experimental import pallas as pl


def kernel(output, target):
    raise NotImplementedError("write your pallas kernel here")


# speedup vs baseline: 3.4945x; 3.4945x over previous
import jax
import jax.numpy as jnp
from jax import lax
from jax.experimental import pallas as pl

B = 4096
C = 1000
BLK = 512
GRID = B // BLK


def _body(x_ref, t_ref, n_ref, out_ref):
    i = pl.program_id(0)
    x = x_ref[...]
    t = t_ref[0, 0, :].reshape(BLK, 1)
    n = n_ref[0, 0, :].reshape(BLK, 1)

    cols = lax.broadcasted_iota(jnp.int32, (BLK, C), 1)
    my = cols == t
    mk = cols == n

    m = jnp.max(x, axis=1, keepdims=True)
    e = jnp.exp(x - m)
    z = jnp.sum(e, axis=1, keepdims=True)

    ey = jnp.sum(jnp.where(my, e, 0.0), axis=1, keepdims=True)
    ek = jnp.sum(jnp.where(mk, e, 0.0), axis=1, keepdims=True)
    oy = jnp.sum(jnp.where(my, x, 0.0), axis=1, keepdims=True)
    ok = jnp.sum(jnp.where(mk, x, 0.0), axis=1, keepdims=True)

    wy = jnp.clip(ey / z, 1e-7, 1.0)
    wk = jnp.clip(ek / z, 1e-7, 1.0)

    tt = 1.0 - (wk - wy)
    gneg = -(wk * (wy + wk)) * tt - wk * (1.0 - wk) * tt
    gpos = wk * tt + wk * wy * tt
    partial = jnp.sum(gneg * ok + gpos * oy).reshape(1, 1)

    prev = jnp.where(i == 0, jnp.zeros((1, 1), jnp.float32), out_ref[...])
    tot = prev + partial
    out_ref[...] = jnp.where(i == GRID - 1, -tot / B, tot)


def kernel(output, target):
    offset = jax.random.randint(jax.random.key(42), (B,), 1, C, dtype=jnp.int32)
    neg = (target + offset) % C
    t3 = target.reshape(GRID, 1, BLK)
    n3 = neg.reshape(GRID, 1, BLK)
    out = pl.pallas_call(
        _body,
        grid=(GRID,),
        in_specs=[
            pl.BlockSpec((BLK, C), lambda i: (i, 0)),
            pl.BlockSpec((1, 1, BLK), lambda i: (i, 0, 0)),
            pl.BlockSpec((1, 1, BLK), lambda i: (i, 0, 0)),
        ],
        out_specs=pl.BlockSpec((1, 1), lambda i: (0, 0)),
        out_shape=jax.ShapeDtypeStruct((1, 1), jnp.float32),
    )(output, t3, n3)
    return out[0, 0]
